# baseline (speedup 1.0000x reference)
import functools

import jax
import jax.numpy as jnp
from jax.experimental import pallas as pl
from jax.experimental.pallas import tpu as pltpu

NUM_EMBEDDING = 8192
EMBEDDING_DIM = 32
COMMITEMENT_COST = 0.25


def _vq_kernel(x_ref, w_ref, x2_ref, w2_ref, enc_ref, q_ref, loss_ref,
               perp_ref, acc_loss, acc_counts, *, num_tiles, n_tokens):
    i = pl.program_id(0)
    t = x_ref.shape[0]
    k = w_ref.shape[0]

    x = x_ref[...]
    w = w_ref[...]

    mm = jax.lax.dot_general(
        x, w, (((1,), (1,)), ((), ())),
        preferred_element_type=jnp.float32)
    x2 = x2_ref[...]
    w2 = w2_ref[...]
    dist = (x2 + w2) - 2.0 * mm

    h = k // 2
    dist_a = dist[:, :h]
    dist_b = dist[:, h:]
    col_h = jax.lax.broadcasted_iota(jnp.int32, (t, h), 1)
    min_a = jnp.min(dist_a, axis=1, keepdims=True)
    idx_a = jnp.min(jnp.where(dist_a == min_a, col_h, h), axis=1, keepdims=True)
    min_b = jnp.min(dist_b, axis=1, keepdims=True)
    idx_b = jnp.min(jnp.where(dist_b == min_b, col_h, h), axis=1, keepdims=True) + h
    s_a = min_a.astype(jnp.bfloat16).astype(jnp.float32)
    idx = jnp.where(min_b < s_a, idx_b, idx_a)

    col = jax.lax.broadcasted_iota(jnp.int32, (t, k), 1)
    onehot = (col == idx).astype(jnp.float32)
    enc_ref[...] = onehot

    q = jax.lax.dot_general(
        onehot, w, (((1,), (0,)), ((), ())),
        preferred_element_type=jnp.float32)
    q_ref[...] = q

    diff = q - x
    part = jnp.sum(diff * diff)
    cnt = jnp.sum(onehot, axis=0, keepdims=True)

    @pl.when(i == 0)
    def _init():
        acc_loss[0, 0] = part
        acc_counts[...] = cnt

    @pl.when(i > 0)
    def _acc():
        acc_loss[0, 0] += part
        acc_counts[...] += cnt

    @pl.when(i == num_tiles - 1)
    def _fini():
        mean_sq = acc_loss[0, 0] / (n_tokens * EMBEDDING_DIM)
        loss_ref[...] = jnp.reshape(mean_sq + COMMITEMENT_COST * mean_sq, (1, 1))
        avg = acc_counts[...] * (1.0 / n_tokens)
        ent = avg * jnp.log(avg + 1e-10)
        perp_ref[...] = jnp.reshape(jnp.exp(-jnp.sum(ent)), (1, 1))


def kernel(inputs, weight):
    B, C, H, W = inputs.shape
    K, D = weight.shape
    n = B * H * W
    x = jnp.transpose(inputs, (0, 2, 3, 1)).reshape(n, D)
    x2 = x[:, 0] * x[:, 0]
    for _d in range(1, D):
        x2 = x2 + x[:, _d] * x[:, _d]
    x2 = x2[:, None]
    w2 = jnp.sum(weight ** 2, axis=1)[None, :]

    tile = 64
    num_tiles = n // tile

    enc, q, loss, perp = pl.pallas_call(
        functools.partial(_vq_kernel, num_tiles=num_tiles, n_tokens=n),
        grid=(num_tiles,),
        in_specs=[
            pl.BlockSpec((tile, D), lambda i: (i, 0)),
            pl.BlockSpec((K, D), lambda i: (0, 0)),
            pl.BlockSpec((tile, 1), lambda i: (i, 0)),
            pl.BlockSpec((1, K), lambda i: (0, 0)),
        ],
        out_specs=[
            pl.BlockSpec((tile, K), lambda i: (i, 0)),
            pl.BlockSpec((tile, D), lambda i: (i, 0)),
            pl.BlockSpec((1, 1), lambda i: (0, 0)),
            pl.BlockSpec((1, 1), lambda i: (0, 0)),
        ],
        out_shape=[
            jax.ShapeDtypeStruct((n, K), jnp.float32),
            jax.ShapeDtypeStruct((n, D), jnp.float32),
            jax.ShapeDtypeStruct((1, 1), jnp.float32),
            jax.ShapeDtypeStruct((1, 1), jnp.float32),
        ],
        scratch_shapes=[
            pltpu.SMEM((1, 1), jnp.float32),
            pltpu.VMEM((1, K), jnp.float32),
        ],
        compiler_params=pltpu.CompilerParams(
            vmem_limit_bytes=100 * 1024 * 1024),
    )(x, weight, x2, w2)

    quantized = jnp.transpose(q.reshape(B, H, W, C), (0, 3, 1, 2))
    return (loss.reshape(()), quantized, perp.reshape(()), enc)


# speedup vs baseline: 1.4208x; 1.4208x over previous
import functools

import jax
import jax.numpy as jnp
from jax.experimental import pallas as pl
from jax.experimental.pallas import tpu as pltpu

NUM_EMBEDDING = 8192
EMBEDDING_DIM = 32
COMMITEMENT_COST = 0.25


def _vq_kernel(x_ref, w_ref, x2_ref, w2_ref, enc_ref, q_ref, loss_ref,
               perp_ref, acc_loss, acc_counts, *, num_tiles, n_tokens):
    i = pl.program_id(0)
    t = x_ref.shape[0]
    k = w_ref.shape[0]

    x = x_ref[...]
    w = w_ref[...]

    mm = jax.lax.dot_general(
        x, w, (((1,), (1,)), ((), ())),
        preferred_element_type=jnp.float32)
    x2 = x2_ref[...]
    w2 = w2_ref[...]
    dist = (x2 + w2) - 2.0 * mm

    h = k // 2
    dist_a = dist[:, :h]
    dist_b = dist[:, h:]
    min_a = jnp.min(dist_a, axis=1, keepdims=True)
    idx_a = jnp.argmin(dist_a, axis=1)[:, None].astype(jnp.int32)
    min_b = jnp.min(dist_b, axis=1, keepdims=True)
    idx_b = jnp.argmin(dist_b, axis=1)[:, None].astype(jnp.int32) + h
    s_a = min_a.astype(jnp.bfloat16).astype(jnp.float32)
    idx = jnp.where(min_b < s_a, idx_b, idx_a)

    col = jax.lax.broadcasted_iota(jnp.int32, (t, k), 1)
    onehot = (col == idx).astype(jnp.float32)
    enc_ref[...] = onehot

    q = jax.lax.dot_general(
        onehot, w, (((1,), (0,)), ((), ())),
        preferred_element_type=jnp.float32)
    q_ref[...] = q

    diff = q - x
    part = jnp.sum(diff * diff)
    cnt = jnp.sum(onehot, axis=0, keepdims=True)

    @pl.when(i == 0)
    def _init():
        acc_loss[0, 0] = part
        acc_counts[...] = cnt

    @pl.when(i > 0)
    def _acc():
        acc_loss[0, 0] += part
        acc_counts[...] += cnt

    @pl.when(i == num_tiles - 1)
    def _fini():
        mean_sq = acc_loss[0, 0] / (n_tokens * EMBEDDING_DIM)
        loss_ref[...] = jnp.reshape(mean_sq + COMMITEMENT_COST * mean_sq, (1, 1))
        avg = acc_counts[...] * (1.0 / n_tokens)
        ent = avg * jnp.log(avg + 1e-10)
        perp_ref[...] = jnp.reshape(jnp.exp(-jnp.sum(ent)), (1, 1))


def kernel(inputs, weight):
    B, C, H, W = inputs.shape
    K, D = weight.shape
    n = B * H * W
    x = jnp.transpose(inputs, (0, 2, 3, 1)).reshape(n, D)
    x2 = x[:, 0] * x[:, 0]
    for _d in range(1, D):
        x2 = x2 + x[:, _d] * x[:, _d]
    x2 = x2[:, None]
    w2 = jnp.sum(weight ** 2, axis=1)[None, :]

    tile = 128
    num_tiles = n // tile

    enc, q, loss, perp = pl.pallas_call(
        functools.partial(_vq_kernel, num_tiles=num_tiles, n_tokens=n),
        grid=(num_tiles,),
        in_specs=[
            pl.BlockSpec((tile, D), lambda i: (i, 0)),
            pl.BlockSpec((K, D), lambda i: (0, 0)),
            pl.BlockSpec((tile, 1), lambda i: (i, 0)),
            pl.BlockSpec((1, K), lambda i: (0, 0)),
        ],
        out_specs=[
            pl.BlockSpec((tile, K), lambda i: (i, 0)),
            pl.BlockSpec((tile, D), lambda i: (i, 0)),
            pl.BlockSpec((1, 1), lambda i: (0, 0)),
            pl.BlockSpec((1, 1), lambda i: (0, 0)),
        ],
        out_shape=[
            jax.ShapeDtypeStruct((n, K), jnp.float32),
            jax.ShapeDtypeStruct((n, D), jnp.float32),
            jax.ShapeDtypeStruct((1, 1), jnp.float32),
            jax.ShapeDtypeStruct((1, 1), jnp.float32),
        ],
        scratch_shapes=[
            pltpu.SMEM((1, 1), jnp.float32),
            pltpu.VMEM((1, K), jnp.float32),
        ],
        compiler_params=pltpu.CompilerParams(
            vmem_limit_bytes=100 * 1024 * 1024),
    )(x, weight, x2, w2)

    quantized = jnp.transpose(q.reshape(B, H, W, C), (0, 3, 1, 2))
    return (loss.reshape(()), quantized, perp.reshape(()), enc)
